# baseline (speedup 1.0000x reference)
import functools

import jax
import jax.numpy as jnp
from jax import lax
from jax.experimental import pallas as pl
from jax.experimental.pallas import tpu as pltpu
from jax.experimental.pallas import tpu_sc as plsc

B, L, K, D = 4, 4096, 16, 64
BQ = 512
NQ = L // BQ
BIG = 0x7FFFFFFF

NC, NS = 2, 16
NW = NC * NS
CH = 128


def _make_select_body(b0):
    def body(q_ref, all_ref, gidx_ref, dist_ref):
        q = q_ref[0]
        allv = all_ref[0]
        d = jnp.abs(q - allv)
        j = lax.broadcasted_iota(jnp.int32, (BQ, L), 1)
        keys = jnp.bitwise_or(jnp.left_shift(d, 12), j).astype(jnp.float32)
        big = jnp.float32(BIG)
        f = jnp.minimum(keys[:, :L // 2], keys[:, L // 2:])
        mx = jnp.maximum(keys[:, :L // 2], keys[:, L // 2:])
        m = jnp.min(f, axis=1, keepdims=True)
        mins = [m]
        for _ in range(K - 1):
            repl = jnp.where(mx > m, mx, big)
            f = jnp.where(f == m, repl, f)
            m = jnp.min(f, axis=1, keepdims=True)
            mins.append(m)
        packed = jnp.concatenate(mins, axis=1).astype(jnp.int32)
        base = (pl.program_id(0) // NQ + b0) * L
        gidx_ref[0] = jnp.bitwise_and(packed, 4095) + base
        dist_ref[0] = jnp.right_shift(packed, 12).astype(jnp.float32)
    return body


def _select(vals, b0, nb):
    q = vals.reshape(nb * NQ, BQ, 1)
    allv = vals.reshape(nb, 1, L)
    return pl.pallas_call(
        _make_select_body(b0),
        grid=(nb * NQ,),
        in_specs=[
            pl.BlockSpec((1, BQ, 1), lambda g: (g, 0, 0)),
            pl.BlockSpec((1, 1, L), lambda g: (g // NQ, 0, 0)),
        ],
        out_specs=[
            pl.BlockSpec((1, BQ, K), lambda g: (g, 0, 0)),
            pl.BlockSpec((1, BQ, K), lambda g: (g, 0, 0)),
        ],
        out_shape=[
            jax.ShapeDtypeStruct((nb * NQ, BQ, K), jnp.int32),
            jax.ShapeDtypeStruct((nb * NQ, BQ, K), jnp.float32),
        ],
        compiler_params=pltpu.CompilerParams(
            dimension_semantics=("parallel",)),
    )(q, allv)


def _make_gather_body(rows_per_w, nch):
    def body(table_hbm, idx_hbm, out_hbm, idx_v, rows0, rows1, shared,
             sem0, sem1):
        wid = lax.axis_index("s") * NC + lax.axis_index("c")
        base = wid * rows_per_w
        @pl.when(lax.axis_index("s") == 0)
        def _():
            pltpu.sync_copy(table_hbm, shared)

        pltpu.sync_copy(idx_hbm.at[pl.ds(base, rows_per_w)], idx_v)
        plsc.subcore_barrier()

        pltpu.async_copy(shared.at[idx_v.at[pl.ds(0, CH)]], rows0, sem0)

        @pl.loop(0, nch // 2)
        def _(jj):
            j = jj * 2
            pltpu.make_async_copy(
                shared.at[idx_v.at[pl.ds(j * CH, CH)]], rows0, sem0).wait()
            pltpu.async_copy(
                shared.at[idx_v.at[pl.ds((j + 1) * CH, CH)]], rows1, sem1)
            pltpu.sync_copy(rows0, out_hbm.at[pl.ds(base + j * CH, CH)])
            pltpu.make_async_copy(
                shared.at[idx_v.at[pl.ds((j + 1) * CH, CH)]],
                rows1, sem1).wait()

            @pl.when(j + 2 < nch)
            def _():
                pltpu.async_copy(
                    shared.at[idx_v.at[pl.ds((j + 2) * CH, CH)]],
                    rows0, sem0)

            pltpu.sync_copy(rows1, out_hbm.at[pl.ds(base + (j + 1) * CH, CH)])
    return body


def _gather(table, gidx, nrows):
    rows_per_w = nrows // NW
    nch = rows_per_w // CH
    sc_gather = functools.partial(
        pl.kernel,
        mesh=plsc.VectorSubcoreMesh(core_axis_name="c", subcore_axis_name="s"),
        out_type=jax.ShapeDtypeStruct((nrows, D), jnp.float32),
        scratch_types=[
            pltpu.VMEM((rows_per_w,), jnp.int32),
            pltpu.VMEM((CH, D), jnp.float32),
            pltpu.VMEM((CH, D), jnp.float32),
            pltpu.VMEM_SHARED((B * L, D), jnp.float32),
            pltpu.SemaphoreType.DMA,
            pltpu.SemaphoreType.DMA,
        ],
        compiler_params=pltpu.CompilerParams(use_tc_tiling_on_sc=False),
    )(_make_gather_body(rows_per_w, nch))
    return sc_gather(table, gidx.reshape(nrows))


def kernel(index, attr):
    vals = index[..., 0].astype(jnp.int32)
    gidx, dist = _select(vals, 0, B)
    rows = _gather(attr.reshape(B * L, D), gidx, B * L * K)
    index_distance = dist.reshape(B, L, K, 1)
    neighbors_attr = rows.reshape(B, L, K, D)
    return (index_distance, neighbors_attr)


# speedup vs baseline: 1.0408x; 1.0408x over previous
import functools

import jax
import jax.numpy as jnp
from jax import lax
from jax.experimental import pallas as pl
from jax.experimental.pallas import tpu as pltpu
from jax.experimental.pallas import tpu_sc as plsc

B, L, K, D = 4, 4096, 16, 64
BQ = 128
NQ = L // BQ
BIG = 0x7FFFFFFF

NC, NS = 2, 16
NW = NC * NS
CH = 128


def _make_select_body(b0):
    def body(q_ref, all_ref, gidx_ref, dist_ref):
        q = q_ref[0]
        allv = all_ref[0]
        d = jnp.abs(q - allv)
        j = lax.broadcasted_iota(jnp.int32, (BQ, L), 1)
        keys = jnp.bitwise_or(jnp.left_shift(d, 12), j).astype(jnp.float32)
        big = jnp.float32(BIG)
        f = jnp.minimum(keys[:, :L // 2], keys[:, L // 2:])
        mx = jnp.maximum(keys[:, :L // 2], keys[:, L // 2:])
        m = jnp.min(f, axis=1, keepdims=True)
        mins = [m]
        for _ in range(K - 1):
            repl = jnp.where(mx > m, mx, big)
            f = jnp.where(f == m, repl, f)
            m = jnp.min(f, axis=1, keepdims=True)
            mins.append(m)
        packed = jnp.concatenate(mins, axis=1).astype(jnp.int32)
        base = (pl.program_id(0) // NQ + b0) * L
        gidx_ref[0] = jnp.bitwise_and(packed, 4095) + base
        dist_ref[0] = jnp.right_shift(packed, 12).astype(jnp.float32)
    return body


def _select(vals, b0, nb):
    q = vals.reshape(nb * NQ, BQ, 1)
    allv = vals.reshape(nb, 1, L)
    return pl.pallas_call(
        _make_select_body(b0),
        grid=(nb * NQ,),
        in_specs=[
            pl.BlockSpec((1, BQ, 1), lambda g: (g, 0, 0)),
            pl.BlockSpec((1, 1, L), lambda g: (g // NQ, 0, 0)),
        ],
        out_specs=[
            pl.BlockSpec((1, BQ, K), lambda g: (g, 0, 0)),
            pl.BlockSpec((1, BQ, K), lambda g: (g, 0, 0)),
        ],
        out_shape=[
            jax.ShapeDtypeStruct((nb * NQ, BQ, K), jnp.int32),
            jax.ShapeDtypeStruct((nb * NQ, BQ, K), jnp.float32),
        ],
        compiler_params=pltpu.CompilerParams(
            dimension_semantics=("parallel",)),
    )(q, allv)


def _make_gather_body(rows_per_w, nch):
    def body(table_hbm, idx_hbm, out_hbm, idx_v, rows0, rows1, shared,
             sem0, sem1):
        wid = lax.axis_index("s") * NC + lax.axis_index("c")
        base = wid * rows_per_w
        @pl.when(lax.axis_index("s") == 0)
        def _():
            pltpu.sync_copy(table_hbm, shared)

        pltpu.sync_copy(idx_hbm.at[pl.ds(base, rows_per_w)], idx_v)
        plsc.subcore_barrier()

        pltpu.async_copy(shared.at[idx_v.at[pl.ds(0, CH)]], rows0, sem0)

        @pl.loop(0, nch // 2)
        def _(jj):
            j = jj * 2
            pltpu.make_async_copy(
                shared.at[idx_v.at[pl.ds(j * CH, CH)]], rows0, sem0).wait()
            pltpu.async_copy(
                shared.at[idx_v.at[pl.ds((j + 1) * CH, CH)]], rows1, sem1)
            pltpu.sync_copy(rows0, out_hbm.at[pl.ds(base + j * CH, CH)])
            pltpu.make_async_copy(
                shared.at[idx_v.at[pl.ds((j + 1) * CH, CH)]],
                rows1, sem1).wait()

            @pl.when(j + 2 < nch)
            def _():
                pltpu.async_copy(
                    shared.at[idx_v.at[pl.ds((j + 2) * CH, CH)]],
                    rows0, sem0)

            pltpu.sync_copy(rows1, out_hbm.at[pl.ds(base + (j + 1) * CH, CH)])
    return body


def _gather(table, gidx, nrows):
    rows_per_w = nrows // NW
    nch = rows_per_w // CH
    sc_gather = functools.partial(
        pl.kernel,
        mesh=plsc.VectorSubcoreMesh(core_axis_name="c", subcore_axis_name="s"),
        out_type=jax.ShapeDtypeStruct((nrows, D), jnp.float32),
        scratch_types=[
            pltpu.VMEM((rows_per_w,), jnp.int32),
            pltpu.VMEM((CH, D), jnp.float32),
            pltpu.VMEM((CH, D), jnp.float32),
            pltpu.VMEM_SHARED((B * L, D), jnp.float32),
            pltpu.SemaphoreType.DMA,
            pltpu.SemaphoreType.DMA,
        ],
        compiler_params=pltpu.CompilerParams(use_tc_tiling_on_sc=False),
    )(_make_gather_body(rows_per_w, nch))
    return sc_gather(table, gidx.reshape(nrows))


def kernel(index, attr):
    vals = index[..., 0].astype(jnp.int32)
    gidx, dist = _select(vals, 0, B)
    rows = _gather(attr.reshape(B * L, D), gidx, B * L * K)
    index_distance = dist.reshape(B, L, K, 1)
    neighbors_attr = rows.reshape(B, L, K, D)
    return (index_distance, neighbors_attr)
